# baseline (speedup 1.0000x reference)
import jax
import jax.numpy as jnp
from jax import lax
from jax.experimental import pallas as pl
from jax.experimental.pallas import tpu as pltpu
from jax.experimental.pallas import tpu_sc as plsc

B = 1024
IU = 128
DU = 32
DI = 32
H = 50
UID_COL = 64
CORPUS = 100000
NUM_ITEMS = 100

TILE = 2048
NT = 56
CPAD = NT * TILE
NCH = CPAD // 32
NPC = CPAD // 128
NPC_REAL = 784

NC = 2
NS = 16
NW = NC * NS
ROWS_W = B // NW

CCAP = 288
FCAP = 256


def _iota16():
    return lax.iota(jnp.int32, 16)


def _mono_key(v):
    u = lax.bitcast_convert_type(v, jnp.int32)
    m = lax.shift_right_logical(lax.shift_right_arithmetic(u, 31), 1)
    return u ^ m


IMIN = -2147483648


def _b1_of(key):
    return lax.shift_right_arithmetic(key, 24) + 128


def _p16_of(key):
    return lax.shift_right_arithmetic(key, 16) + 32768


def _p24_of(key):
    return lax.shift_right_arithmetic(key, 8) + 8388608


def _bits(key, sh):
    return lax.shift_right_logical(key, sh) & 0xFF


def _haddr(b):
    iota = _iota16()
    return iota * 256 + ((b + iota) & 255)


def _embed_body(idx3_hbm, utab3_hbm, itab3_hbm, emb_out,
                hidx_v, urow_v, urowb_v, hrows_v, hrowsb_v, sel_v,
                sem, semb):
    wid = lax.axis_index("s") * NC + lax.axis_index("c")
    base = wid * ROWS_W
    rg0 = base // 8
    inv_h = jnp.float32(1.0 / H)
    zf16 = jnp.zeros((16,), jnp.float32)

    def group_loop(gi, c):
        pltpu.sync_copy(idx3_hbm.at[rg0 + gi], hidx_v)

        def fire(si, hr_v, ur_v, s2):
            uid = hidx_v[si, pl.ds(UID_COL, 16)][0]
            pltpu.async_copy(utab3_hbm.at[uid >> 3], ur_v, s2)

            def fire_h(j, c3):
                hj = hidx_v[si, pl.ds(j, 16)][0]
                pltpu.async_copy(itab3_hbm.at[hj >> 3], hr_v.at[j], s2)
                return c3

            lax.fori_loop(0, H, fire_h, 0)

        def consume(si, hr_v, ur_v, s2):
            s = gi * 8 + si
            uid = hidx_v[si, pl.ds(UID_COL, 16)][0]
            pltpu.make_async_copy(utab3_hbm.at[0], ur_v, s2).wait()
            pltpu.make_async_copy(itab3_hbm.at[pl.ds(0, H)], hr_v,
                                  s2).wait()
            usub = uid & 7
            sel_v[s, pl.ds(0, 16)] = ur_v[usub, pl.ds(0, 16)]
            sel_v[s, pl.ds(16, 16)] = ur_v[usub, pl.ds(16, 16)]

            def acc_h(j, acc):
                a0, a1 = acc
                sub = hidx_v[si, pl.ds(j, 16)][0] & 7
                a0 = a0 + hr_v[j, sub, pl.ds(0, 16)]
                a1 = a1 + hr_v[j, sub, pl.ds(16, 16)]
                return a0, a1

            a0, a1 = lax.fori_loop(0, H, acc_h, (zf16, zf16))
            sel_v[s, pl.ds(32, 16)] = a0 * inv_h
            sel_v[s, pl.ds(48, 16)] = a1 * inv_h
            sel_v[s, pl.ds(64, 16)] = zf16
            sel_v[s, pl.ds(80, 16)] = zf16
            sel_v[s, pl.ds(96, 16)] = zf16
            sel_v[s, pl.ds(112, 16)] = zf16

        fire(0, hrows_v, urow_v, sem)

        def pair_loop(p, cc):
            s0 = 2 * p
            fire(s0 + 1, hrowsb_v, urowb_v, semb)
            consume(s0, hrows_v, urow_v, sem)

            @pl.when(p < 3)
            def _nf():
                fire(s0 + 2, hrows_v, urow_v, sem)

            consume(s0 + 1, hrowsb_v, urowb_v, semb)
            return cc

        lax.fori_loop(0, 4, pair_loop, 0)
        return c

    lax.fori_loop(0, ROWS_W // 8, group_loop, 0)
    pltpu.sync_copy(sel_v, emb_out.at[pl.ds(base, ROWS_W)])


def _sc_embed(idx3, utab3, itab3):
    mesh = plsc.VectorSubcoreMesh(core_axis_name="c", subcore_axis_name="s")
    fn = pl.kernel(
        _embed_body,
        out_type=jax.ShapeDtypeStruct((B, 128), jnp.float32),
        mesh=mesh,
        compiler_params=pltpu.CompilerParams(needs_layout_passes=False),
        scratch_types=[
            pltpu.VMEM((8, 128), jnp.int32),
            pltpu.VMEM((8, DU), jnp.float32),
            pltpu.VMEM((8, DU), jnp.float32),
            pltpu.VMEM((H, 8, DI), jnp.float32),
            pltpu.VMEM((H, 8, DI), jnp.float32),
            pltpu.VMEM((ROWS_W, 128), jnp.float32),
            pltpu.SemaphoreType.DMA,
            pltpu.SemaphoreType.DMA,
        ],
    )
    return fn(idx3, utab3, itab3)


def _tower_body(uf_ref, emb_ref, w1_ref, b1_ref, w2_ref, b2_ref,
                wt_ref, bt_ref, out_ref):
    h = jnp.maximum(uf_ref[...] @ w1_ref[...] + b1_ref[...], 0.0)
    feat = h @ w2_ref[...] + b2_ref[...]
    x = jnp.concatenate([emb_ref[:, 0:DU], feat, emb_ref[:, DU:2 * DU]],
                        axis=1)
    out_ref[...] = x @ wt_ref[...] + bt_ref[...]


def _scores_body(ue_ref, corpus_ref, scores_ref, cmax_ref):
    j = pl.program_id(0)
    s = lax.dot_general(ue_ref[...], corpus_ref[...],
                        (((1,), (1,)), ((), ())),
                        preferred_element_type=jnp.float32)
    col = j * TILE + lax.broadcasted_iota(jnp.int32, s.shape, 1)
    s = jnp.where(col < CORPUS, s, -jnp.inf)
    scores_ref[...] = s.reshape(B, TILE // 128, 128)
    st = s.T
    cm_t = jnp.max(st.reshape(TILE // 32, 32, B), axis=1)
    cm = cm_t.T

    @pl.when(j % 2 == 0)
    def _even():
        cmax_ref[:, 0:64] = cm

    @pl.when(j % 2 == 1)
    def _odd():
        cmax_ref[:, 64:128] = cm


def _scan_level(hist_v, cum0, kcount):
    zeros16 = jnp.zeros((16,), jnp.int32)
    iota = _iota16()

    def group(t, carry):
        j = 15 - t
        done, bsel, chi, cum = carry
        tot = zeros16
        for l in range(16):
            addr = l * 256 + ((j * 16 + l + iota) & 255)
            tot = tot + plsc.load_gather(hist_v, [addr])
            plsc.store_scatter(hist_v, [addr], zeros16)
        rev = lax.rev(tot, (0,))
        cs = plsc.cumsum(rev) + cum
        m = cs >= kcount
        npos = plsc.all_reduce_population_count(m)
        f = plsc.all_reduce_ffs(m)
        sel = _iota16() == f
        csf = jnp.sum(jnp.where(sel, cs, 0))
        totf = jnp.sum(jnp.where(sel, rev, 0))
        found = jnp.logical_and(npos > 0, jnp.logical_not(done))
        bnew = jnp.full((16,), j * 16 + 15, jnp.int32) - f
        bsel = jnp.where(found, bnew, bsel)
        chi = jnp.where(found, csf - totf, chi)
        done = jnp.logical_or(done, npos > 0)
        cum = jnp.where(done, cum, jnp.max(cs) + jnp.zeros((16,), jnp.int32))
        return done, bsel, chi, cum

    done0 = jnp.zeros((16,), jnp.bool_)
    init = (done0, jnp.zeros((16,), jnp.int32), jnp.zeros((16,), jnp.int32),
            cum0)
    _, bsel, chi, _ = lax.fori_loop(0, 16, group, init)
    return bsel, chi


def _topk_body(scores_hbm, cmax_hbm, out_hbm,
               row_v, cm_v, hist_v, candc_v, ekey_v,
               fkey_v, fidx_v, out_v, sem_row, sem_cm):
    wid = lax.axis_index("s") * NC + lax.axis_index("c")
    base = wid * ROWS_W
    iota = _iota16()
    zeros16 = jnp.zeros((16,), jnp.int32)
    ones16 = jnp.ones((16,), jnp.int32)
    k100 = jnp.int32(NUM_ITEMS)

    def clr(i, c):
        hist_v[pl.ds(i * 16, 16)] = zeros16
        return c

    lax.fori_loop(0, 256, clr, 0)

    def row_step(k, carry):
        r = base + k
        cp_row = pltpu.async_copy(scores_hbm.at[r, pl.ds(0, NPC_REAL)],
                                  row_v, sem_row)
        pltpu.sync_copy(cmax_hbm.at[r], cm_v)


        def l1(i4, c):
            for u in range(4):
                i = i4 * 4 + u
                g = i >> 3
                o = (i & 7) * 16
                key = _mono_key(cm_v[g, pl.ds(o, 16)])
                cm_v[g, pl.ds(o, 16)] = lax.bitcast_convert_type(
                    key, jnp.float32)
                plsc.addupdate_scatter(hist_v, [_haddr(_b1_of(key))],
                                       ones16)
            return c

        lax.fori_loop(0, NCH // 64, l1, 0)
        b1s, chi1 = _scan_level(hist_v, zeros16, k100)

        def l2(i4, c):
            for u in range(4):
                i = i4 * 4 + u
                g = i >> 3
                o = (i & 7) * 16
                key = lax.bitcast_convert_type(cm_v[g, pl.ds(o, 16)],
                                               jnp.int32)
                meq = _b1_of(key) == b1s
                plsc.addupdate_scatter(hist_v, [_haddr(_bits(key, 16))],
                                       ones16, mask=meq)
            return c

        lax.fori_loop(0, NCH // 64, l2, 0)
        b2s, _ = _scan_level(hist_v, chi1, k100)
        t16 = b1s * 256 + b2s

        def coll(i4, off):
            for u in range(4):
                i = i4 * 4 + u
                g = i >> 3
                o = (i & 7) * 16
                key = lax.bitcast_convert_type(cm_v[g, pl.ds(o, 16)],
                                               jnp.int32)
                m = _p16_of(key) >= t16
                pos = plsc.cumsum(m.astype(jnp.int32)) + off - 1
                m2 = jnp.logical_and(m, pos < CCAP)
                plsc.store_scatter(candc_v, [pos], iota + i * 16, mask=m2)
                off = off + plsc.all_reduce_population_count(m)
            return off

        offc = lax.fori_loop(0, NCH // 64, coll, zeros16)
        ns = jnp.minimum(jnp.max(offc), jnp.int32(CCAP))

        cp_row.wait()

        def ext(s2i, c):
            for u in range(2):
                s = s2i * 2 + u
                cid = candc_v[pl.ds(s, 16)][0]
                p = cid >> 2
                o = (cid & 3) * 32
                key0 = _mono_key(row_v[p, pl.ds(o, 16)])
                key1 = _mono_key(row_v[p, pl.ds(o + 16, 16)])
                ekey_v[pl.ds(s * 32, 16)] = key0
                ekey_v[pl.ds(s * 32 + 16, 16)] = key1
            return c

        lax.fori_loop(0, (ns + 1) // 2, ext, 0)
        ns2 = ns * 2
        imin16 = jnp.full((16,), IMIN, jnp.int32)
        ekey_v[pl.ds(ns2 * 16, 16)] = imin16
        ekey_v[pl.ds(ns2 * 16 + 16, 16)] = imin16
        nsu = (ns2 + 1) // 2

        def e1(s2i, c):
            for u in range(2):
                s = s2i * 2 + u
                key = ekey_v[pl.ds(s * 16, 16)]
                plsc.addupdate_scatter(hist_v, [_haddr(_b1_of(key))],
                                       ones16)
            return c

        lax.fori_loop(0, nsu, e1, 0)
        eb1, echi1 = _scan_level(hist_v, zeros16, k100)

        def e2(s2i, c):
            for u in range(2):
                s = s2i * 2 + u
                key = ekey_v[pl.ds(s * 16, 16)]
                meq = _b1_of(key) == eb1
                plsc.addupdate_scatter(hist_v, [_haddr(_bits(key, 16))],
                                       ones16, mask=meq)
            return c

        lax.fori_loop(0, nsu, e2, 0)
        eb2, echi2 = _scan_level(hist_v, echi1, k100)
        et16 = eb1 * 256 + eb2

        def e3(s2i, c):
            for u in range(2):
                s = s2i * 2 + u
                key = ekey_v[pl.ds(s * 16, 16)]
                meq = _p16_of(key) == et16
                plsc.addupdate_scatter(hist_v, [_haddr(_bits(key, 8))],
                                       ones16, mask=meq)
            return c

        lax.fori_loop(0, nsu, e3, 0)
        eb3, _ = _scan_level(hist_v, echi2, k100)
        et24 = eb1 * 65536 + eb2 * 256 + eb3

        def collf(s2i, off):
            for u in range(2):
                s = s2i * 2 + u
                key = ekey_v[pl.ds(s * 16, 16)]
                cid = candc_v[pl.ds(s >> 1, 16)][0]
                ei = iota + cid * 32 + (s & 1) * 16
                m = _p24_of(key) >= et24
                pos = plsc.cumsum(m.astype(jnp.int32)) + off - 1
                m2 = jnp.logical_and(m, pos < FCAP)
                plsc.store_scatter(fkey_v, [pos], key, mask=m2)
                plsc.store_scatter(fidx_v, [pos], ei, mask=m2)
                off = off + plsc.all_reduce_population_count(m)
            return off

        offf = lax.fori_loop(0, nsu, collf, zeros16)
        nf = jnp.minimum(jnp.max(offf), jnp.int32(FCAP))
        padpos = nf + iota
        plsc.store_scatter(fkey_v, [padpos], zeros16 + IMIN,
                           mask=padpos < FCAP)
        nfv = (nf + jnp.int32(15)) // jnp.int32(16)

        def rank_group(g, c):
            kg = fkey_v[pl.ds(g * 16, 16)]
            ig = fidx_v[pl.ds(g * 16, 16)]

            def rank_elem(e, cc):
                sel = iota == e
                ke = jnp.full((16,), jnp.sum(jnp.where(sel, kg, 0)),
                              jnp.int32)
                ie = jnp.full((16,), jnp.sum(jnp.where(sel, ig, 0)),
                              jnp.int32)

                def cnt(g2, rk):
                    k2 = fkey_v[pl.ds(g2 * 16, 16)]
                    i2 = fidx_v[pl.ds(g2 * 16, 16)]
                    gt = k2 > ke
                    eq = jnp.logical_and(k2 == ke, i2 < ie)
                    return (rk + plsc.all_reduce_population_count(gt)
                            + plsc.all_reduce_population_count(eq))

                rk = lax.fori_loop(0, nfv, cnt, zeros16)
                mout = jnp.logical_and(jnp.logical_and(rk < 100, sel),
                                       ke != IMIN)
                plsc.store_scatter(out_v, [zeros16, rk], ie, mask=mout)
                return cc

            lax.fori_loop(0, 16, rank_elem, 0)
            return c

        lax.fori_loop(0, nfv, rank_group, 0)
        pltpu.sync_copy(out_v, out_hbm.at[r])
        return carry

    lax.fori_loop(0, ROWS_W, row_step, 0)


def _sc_topk(scores, cmax):
    mesh = plsc.VectorSubcoreMesh(core_axis_name="c", subcore_axis_name="s")
    fn = pl.kernel(
        _topk_body,
        out_type=jax.ShapeDtypeStruct((B, 8, 128), jnp.int32),
        mesh=mesh,
        compiler_params=pltpu.CompilerParams(needs_layout_passes=False,
                                             use_tc_tiling_on_sc=True),
        scratch_types=[
            pltpu.VMEM((NPC_REAL, 128), jnp.float32),
            pltpu.VMEM((NCH // 128, 128), jnp.float32),
            pltpu.VMEM((4096,), jnp.int32),
            pltpu.VMEM((CCAP + 16,), jnp.int32),
            pltpu.VMEM((CCAP * 32 + 32,), jnp.int32),
            pltpu.VMEM((FCAP,), jnp.int32),
            pltpu.VMEM((FCAP,), jnp.int32),
            pltpu.VMEM((8, 128), jnp.int32),
            pltpu.SemaphoreType.DMA,
            pltpu.SemaphoreType.DMA,
        ],
    )
    return fn(scores, cmax)


def kernel(user_id, user_features, user_history, user_table, W1, b1, W2, b2,
           Wt, bt, item_table, corpus_embeddings):
    idx3 = jnp.concatenate(
        [user_history.astype(jnp.int32),
         jnp.zeros((B, UID_COL - H), jnp.int32),
         user_id.reshape(B, 1).astype(jnp.int32),
         jnp.zeros((B, 127 - UID_COL), jnp.int32)],
        axis=1).reshape(B // 8, 8, 128)
    emb = _sc_embed(idx3, user_table.reshape(-1, 8, DU),
                    item_table.reshape(-1, 8, DI))

    user_embedding = pl.pallas_call(
        _tower_body,
        out_shape=jax.ShapeDtypeStruct((B, DI), jnp.float32),
    )(user_features, emb, W1, b1.reshape(1, -1), W2,
      b2.reshape(1, -1), Wt, bt.reshape(1, -1))

    corpus_pad = jnp.pad(corpus_embeddings, ((0, CPAD - CORPUS), (0, 0)))
    scores, cmax = pl.pallas_call(
        _scores_body,
        grid=(NT,),
        in_specs=[
            pl.BlockSpec((B, DU), lambda j: (0, 0)),
            pl.BlockSpec((TILE, DU), lambda j: (j, 0)),
        ],
        out_specs=[
            pl.BlockSpec((B, TILE // 128, 128), lambda j: (0, j, 0)),
            pl.BlockSpec((B, 128), lambda j: (0, j // 2)),
        ],
        out_shape=[
            jax.ShapeDtypeStruct((B, NPC, 128), jnp.float32),
            jax.ShapeDtypeStruct((B, NCH), jnp.float32),
        ],
    )(user_embedding, corpus_pad)

    top = _sc_topk(scores, cmax.reshape(B, NCH // 128, 128))
    return top[:, 0, :NUM_ITEMS]


# speedup vs baseline: 1.0058x; 1.0058x over previous
import jax
import jax.numpy as jnp
from jax import lax
from jax.experimental import pallas as pl
from jax.experimental.pallas import tpu as pltpu
from jax.experimental.pallas import tpu_sc as plsc

B = 1024
IU = 128
DU = 32
DI = 32
H = 50
UID_COL = 64
CORPUS = 100000
NUM_ITEMS = 100

TILE = 2048
NT = 56
CPAD = NT * TILE
NCH = CPAD // 32
NPC = CPAD // 128
NPC_REAL = 784

NC = 2
NS = 16
NW = NC * NS
ROWS_W = B // NW

CCAP = 288
FCAP = 256


def _iota16():
    return lax.iota(jnp.int32, 16)


def _mono_key(v):
    u = lax.bitcast_convert_type(v, jnp.int32)
    m = lax.shift_right_logical(lax.shift_right_arithmetic(u, 31), 1)
    return u ^ m


IMIN = -2147483648


def _b1_of(key):
    return lax.shift_right_arithmetic(key, 24) + 128


def _p16_of(key):
    return lax.shift_right_arithmetic(key, 16) + 32768


def _p24_of(key):
    return lax.shift_right_arithmetic(key, 8) + 8388608


def _bits(key, sh):
    return lax.shift_right_logical(key, sh) & 0xFF


def _haddr(b):
    iota = _iota16()
    return iota * 256 + ((b + iota) & 255)


def _embed_body(idx3_hbm, utab3_hbm, itab3_hbm, emb_out,
                hidx_v, urow_v, urowb_v, hrows_v, hrowsb_v, sel_v,
                sem, semb):
    wid = lax.axis_index("s") * NC + lax.axis_index("c")
    base = wid * ROWS_W
    rg0 = base // 8
    inv_h = jnp.float32(1.0 / H)
    zf16 = jnp.zeros((16,), jnp.float32)

    def group_loop(gi, c):
        pltpu.sync_copy(idx3_hbm.at[rg0 + gi], hidx_v)

        def fire(si, hr_v, ur_v, s2):
            uid = hidx_v[si, pl.ds(UID_COL, 16)][0]
            pltpu.async_copy(utab3_hbm.at[uid >> 3], ur_v, s2)

            def fire_h(j, c3):
                hj = hidx_v[si, pl.ds(j, 16)][0]
                pltpu.async_copy(itab3_hbm.at[hj >> 3], hr_v.at[j], s2)
                return c3

            lax.fori_loop(0, H, fire_h, 0)

        def consume(si, hr_v, ur_v, s2):
            s = gi * 8 + si
            uid = hidx_v[si, pl.ds(UID_COL, 16)][0]
            pltpu.make_async_copy(utab3_hbm.at[0], ur_v, s2).wait()
            pltpu.make_async_copy(itab3_hbm.at[pl.ds(0, H)], hr_v,
                                  s2).wait()
            usub = uid & 7
            sel_v[s, pl.ds(0, 16)] = ur_v[usub, pl.ds(0, 16)]
            sel_v[s, pl.ds(16, 16)] = ur_v[usub, pl.ds(16, 16)]

            def acc_h(j, acc):
                a0, a1 = acc
                sub = hidx_v[si, pl.ds(j, 16)][0] & 7
                a0 = a0 + hr_v[j, sub, pl.ds(0, 16)]
                a1 = a1 + hr_v[j, sub, pl.ds(16, 16)]
                return a0, a1

            a0, a1 = lax.fori_loop(0, H, acc_h, (zf16, zf16))
            sel_v[s, pl.ds(32, 16)] = a0 * inv_h
            sel_v[s, pl.ds(48, 16)] = a1 * inv_h
            sel_v[s, pl.ds(64, 16)] = zf16
            sel_v[s, pl.ds(80, 16)] = zf16
            sel_v[s, pl.ds(96, 16)] = zf16
            sel_v[s, pl.ds(112, 16)] = zf16

        fire(0, hrows_v, urow_v, sem)

        def pair_loop(p, cc):
            s0 = 2 * p
            fire(s0 + 1, hrowsb_v, urowb_v, semb)
            consume(s0, hrows_v, urow_v, sem)

            @pl.when(p < 3)
            def _nf():
                fire(s0 + 2, hrows_v, urow_v, sem)

            consume(s0 + 1, hrowsb_v, urowb_v, semb)
            return cc

        lax.fori_loop(0, 4, pair_loop, 0)
        return c

    lax.fori_loop(0, ROWS_W // 8, group_loop, 0)
    pltpu.sync_copy(sel_v, emb_out.at[pl.ds(base, ROWS_W)])


def _sc_embed(idx3, utab3, itab3):
    mesh = plsc.VectorSubcoreMesh(core_axis_name="c", subcore_axis_name="s")
    fn = pl.kernel(
        _embed_body,
        out_type=jax.ShapeDtypeStruct((B, 128), jnp.float32),
        mesh=mesh,
        compiler_params=pltpu.CompilerParams(needs_layout_passes=False),
        scratch_types=[
            pltpu.VMEM((8, 128), jnp.int32),
            pltpu.VMEM((8, DU), jnp.float32),
            pltpu.VMEM((8, DU), jnp.float32),
            pltpu.VMEM((H, 8, DI), jnp.float32),
            pltpu.VMEM((H, 8, DI), jnp.float32),
            pltpu.VMEM((ROWS_W, 128), jnp.float32),
            pltpu.SemaphoreType.DMA,
            pltpu.SemaphoreType.DMA,
        ],
    )
    return fn(idx3, utab3, itab3)


def _tower_body(uf_ref, emb_ref, w1_ref, b1_ref, w2_ref, b2_ref,
                wt_ref, bt_ref, out_ref):
    h = jnp.maximum(uf_ref[...] @ w1_ref[...] + b1_ref[...], 0.0)
    feat = h @ w2_ref[...] + b2_ref[...]
    x = jnp.concatenate([emb_ref[:, 0:DU], feat, emb_ref[:, DU:2 * DU]],
                        axis=1)
    out_ref[...] = x @ wt_ref[...] + bt_ref[...]


def _scores_body(ue_ref, corpus_ref, scores_ref, cmax_ref):
    j = pl.program_id(0)
    s = lax.dot_general(ue_ref[...], corpus_ref[...],
                        (((1,), (1,)), ((), ())),
                        preferred_element_type=jnp.float32)
    col = j * TILE + lax.broadcasted_iota(jnp.int32, s.shape, 1)
    s = jnp.where(col < CORPUS, s, -jnp.inf)
    scores_ref[...] = s.reshape(B, TILE // 128, 128)
    st = s.T
    cm_t = jnp.max(st.reshape(TILE // 32, 32, B), axis=1)
    cm = cm_t.T

    @pl.when(j % 2 == 0)
    def _even():
        cmax_ref[:, 0:64] = cm

    @pl.when(j % 2 == 1)
    def _odd():
        cmax_ref[:, 64:128] = cm


def _scan_level(hist_v, cum0, kcount):
    zeros16 = jnp.zeros((16,), jnp.int32)
    iota = _iota16()

    def group(t, carry):
        j = 15 - t
        done, bsel, chi, cum = carry
        tot = zeros16
        for l in range(16):
            addr = l * 256 + ((j * 16 + l + iota) & 255)
            tot = tot + plsc.load_gather(hist_v, [addr])
            plsc.store_scatter(hist_v, [addr], zeros16)
        rev = lax.rev(tot, (0,))
        cs = plsc.cumsum(rev) + cum
        m = cs >= kcount
        npos = plsc.all_reduce_population_count(m)
        f = plsc.all_reduce_ffs(m)
        sel = _iota16() == f
        csf = jnp.sum(jnp.where(sel, cs, 0))
        totf = jnp.sum(jnp.where(sel, rev, 0))
        found = jnp.logical_and(npos > 0, jnp.logical_not(done))
        bnew = jnp.full((16,), j * 16 + 15, jnp.int32) - f
        bsel = jnp.where(found, bnew, bsel)
        chi = jnp.where(found, csf - totf, chi)
        done = jnp.logical_or(done, npos > 0)
        cum = jnp.where(done, cum, jnp.max(cs) + jnp.zeros((16,), jnp.int32))
        return done, bsel, chi, cum

    done0 = jnp.zeros((16,), jnp.bool_)
    init = (done0, jnp.zeros((16,), jnp.int32), jnp.zeros((16,), jnp.int32),
            cum0)
    _, bsel, chi, _ = lax.fori_loop(0, 16, group, init)
    return bsel, chi


def _topk_body(scores_hbm, cmax_hbm, out_hbm,
               row_v, cm_v, hist_v, candc_v, ekey_v,
               fkey_v, fidx_v, out_v, sem_row, sem_cm):
    wid = lax.axis_index("s") * NC + lax.axis_index("c")
    base = wid * ROWS_W
    iota = _iota16()
    zeros16 = jnp.zeros((16,), jnp.int32)
    ones16 = jnp.ones((16,), jnp.int32)
    k100 = jnp.int32(NUM_ITEMS)

    def clr(i, c):
        hist_v[pl.ds(i * 16, 16)] = zeros16
        return c

    lax.fori_loop(0, 256, clr, 0)

    def row_step(k, carry):
        r = base + k
        cp_row = pltpu.async_copy(scores_hbm.at[r, pl.ds(0, NPC_REAL)],
                                  row_v, sem_row)
        pltpu.sync_copy(cmax_hbm.at[r], cm_v)


        def l1(i4, c):
            for u in range(4):
                i = i4 * 4 + u
                g = i >> 3
                o = (i & 7) * 16
                key = _mono_key(cm_v[g, pl.ds(o, 16)])
                cm_v[g, pl.ds(o, 16)] = lax.bitcast_convert_type(
                    key, jnp.float32)
                plsc.addupdate_scatter(hist_v, [_haddr(_b1_of(key))],
                                       ones16)
            return c

        lax.fori_loop(0, NCH // 64, l1, 0)
        b1s, chi1 = _scan_level(hist_v, zeros16, k100)

        def l2(i4, c):
            for u in range(4):
                i = i4 * 4 + u
                g = i >> 3
                o = (i & 7) * 16
                key = lax.bitcast_convert_type(cm_v[g, pl.ds(o, 16)],
                                               jnp.int32)
                meq = _b1_of(key) == b1s
                plsc.addupdate_scatter(hist_v, [_haddr(_bits(key, 16))],
                                       ones16, mask=meq)
            return c

        lax.fori_loop(0, NCH // 64, l2, 0)
        b2s, _ = _scan_level(hist_v, chi1, k100)
        t16 = b1s * 256 + b2s

        def coll(i4, off):
            for u in range(4):
                i = i4 * 4 + u
                g = i >> 3
                o = (i & 7) * 16
                key = lax.bitcast_convert_type(cm_v[g, pl.ds(o, 16)],
                                               jnp.int32)
                m = _p16_of(key) >= t16
                pos = plsc.cumsum(m.astype(jnp.int32)) + off - 1
                m2 = jnp.logical_and(m, pos < CCAP)
                plsc.store_scatter(candc_v, [pos], iota + i * 16, mask=m2)
                off = off + plsc.all_reduce_population_count(m)
            return off

        offc = lax.fori_loop(0, NCH // 64, coll, zeros16)
        ns = jnp.minimum(jnp.max(offc), jnp.int32(CCAP))

        cp_row.wait()

        def ext(s, c):
            cid = candc_v[pl.ds(s, 16)][0]
            p = cid >> 2
            o = (cid & 3) * 32
            key0 = _mono_key(row_v[p, pl.ds(o, 16)])
            key1 = _mono_key(row_v[p, pl.ds(o + 16, 16)])
            ekey_v[pl.ds(s * 32, 16)] = key0
            ekey_v[pl.ds(s * 32 + 16, 16)] = key1
            return c

        lax.fori_loop(0, ns, ext, 0)
        ns2 = ns * 2

        def e1(s, c):
            key = ekey_v[pl.ds(s * 16, 16)]
            plsc.addupdate_scatter(hist_v, [_haddr(_b1_of(key))], ones16)
            return c

        lax.fori_loop(0, ns2, e1, 0)
        eb1, echi1 = _scan_level(hist_v, zeros16, k100)

        def e2(s, c):
            key = ekey_v[pl.ds(s * 16, 16)]
            meq = _b1_of(key) == eb1
            plsc.addupdate_scatter(hist_v, [_haddr(_bits(key, 16))],
                                   ones16, mask=meq)
            return c

        lax.fori_loop(0, ns2, e2, 0)
        eb2, echi2 = _scan_level(hist_v, echi1, k100)
        et16 = eb1 * 256 + eb2

        def e3(s, c):
            key = ekey_v[pl.ds(s * 16, 16)]
            meq = _p16_of(key) == et16
            plsc.addupdate_scatter(hist_v, [_haddr(_bits(key, 8))],
                                   ones16, mask=meq)
            return c

        lax.fori_loop(0, ns2, e3, 0)
        eb3, _ = _scan_level(hist_v, echi2, k100)
        et24 = eb1 * 65536 + eb2 * 256 + eb3

        def collf(s, off):
            key = ekey_v[pl.ds(s * 16, 16)]
            cid = candc_v[pl.ds(s >> 1, 16)][0]
            ei = iota + cid * 32 + (s & 1) * 16
            m = _p24_of(key) >= et24
            pos = plsc.cumsum(m.astype(jnp.int32)) + off - 1
            m2 = jnp.logical_and(m, pos < FCAP)
            plsc.store_scatter(fkey_v, [pos], key, mask=m2)
            plsc.store_scatter(fidx_v, [pos], ei, mask=m2)
            return off + plsc.all_reduce_population_count(m)

        offf = lax.fori_loop(0, ns2, collf, zeros16)
        nf = jnp.minimum(jnp.max(offf), jnp.int32(FCAP))
        padpos = nf + iota
        plsc.store_scatter(fkey_v, [padpos], zeros16 + IMIN,
                           mask=padpos < FCAP)
        nfv = (nf + jnp.int32(15)) // jnp.int32(16)

        def rank_group(g, c):
            kg = fkey_v[pl.ds(g * 16, 16)]
            ig = fidx_v[pl.ds(g * 16, 16)]

            def rank_elem(e, cc):
                sel = iota == e
                ke = jnp.full((16,), jnp.sum(jnp.where(sel, kg, 0)),
                              jnp.int32)
                ie = jnp.full((16,), jnp.sum(jnp.where(sel, ig, 0)),
                              jnp.int32)

                def cnt(g2, rk):
                    k2 = fkey_v[pl.ds(g2 * 16, 16)]
                    i2 = fidx_v[pl.ds(g2 * 16, 16)]
                    gt = k2 > ke
                    eq = jnp.logical_and(k2 == ke, i2 < ie)
                    return (rk + plsc.all_reduce_population_count(gt)
                            + plsc.all_reduce_population_count(eq))

                rk = lax.fori_loop(0, nfv, cnt, zeros16)
                mout = jnp.logical_and(jnp.logical_and(rk < 100, sel),
                                       ke != IMIN)
                plsc.store_scatter(out_v, [zeros16, rk], ie, mask=mout)
                return cc

            lax.fori_loop(0, 16, rank_elem, 0)
            return c

        lax.fori_loop(0, nfv, rank_group, 0)
        pltpu.sync_copy(out_v, out_hbm.at[r])
        return carry

    lax.fori_loop(0, ROWS_W, row_step, 0)


def _sc_topk(scores, cmax):
    mesh = plsc.VectorSubcoreMesh(core_axis_name="c", subcore_axis_name="s")
    fn = pl.kernel(
        _topk_body,
        out_type=jax.ShapeDtypeStruct((B, 8, 128), jnp.int32),
        mesh=mesh,
        compiler_params=pltpu.CompilerParams(needs_layout_passes=False,
                                             use_tc_tiling_on_sc=True),
        scratch_types=[
            pltpu.VMEM((NPC_REAL, 128), jnp.float32),
            pltpu.VMEM((NCH // 128, 128), jnp.float32),
            pltpu.VMEM((4096,), jnp.int32),
            pltpu.VMEM((CCAP + 16,), jnp.int32),
            pltpu.VMEM((CCAP * 32,), jnp.int32),
            pltpu.VMEM((FCAP,), jnp.int32),
            pltpu.VMEM((FCAP,), jnp.int32),
            pltpu.VMEM((8, 128), jnp.int32),
            pltpu.SemaphoreType.DMA,
            pltpu.SemaphoreType.DMA,
        ],
    )
    return fn(scores, cmax)


def kernel(user_id, user_features, user_history, user_table, W1, b1, W2, b2,
           Wt, bt, item_table, corpus_embeddings):
    idx3 = jnp.concatenate(
        [user_history.astype(jnp.int32),
         jnp.zeros((B, UID_COL - H), jnp.int32),
         user_id.reshape(B, 1).astype(jnp.int32),
         jnp.zeros((B, 127 - UID_COL), jnp.int32)],
        axis=1).reshape(B // 8, 8, 128)
    emb = _sc_embed(idx3, user_table.reshape(-1, 8, DU),
                    item_table.reshape(-1, 8, DI))

    user_embedding = pl.pallas_call(
        _tower_body,
        out_shape=jax.ShapeDtypeStruct((B, DI), jnp.float32),
    )(user_features, emb, W1, b1.reshape(1, -1), W2,
      b2.reshape(1, -1), Wt, bt.reshape(1, -1))

    corpus_pad = jnp.pad(corpus_embeddings, ((0, CPAD - CORPUS), (0, 0)))
    scores, cmax = pl.pallas_call(
        _scores_body,
        grid=(NT,),
        in_specs=[
            pl.BlockSpec((B, DU), lambda j: (0, 0)),
            pl.BlockSpec((TILE, DU), lambda j: (j, 0)),
        ],
        out_specs=[
            pl.BlockSpec((B, TILE // 128, 128), lambda j: (0, j, 0)),
            pl.BlockSpec((B, 128), lambda j: (0, j // 2)),
        ],
        out_shape=[
            jax.ShapeDtypeStruct((B, NPC, 128), jnp.float32),
            jax.ShapeDtypeStruct((B, NCH), jnp.float32),
        ],
    )(user_embedding, corpus_pad)

    top = _sc_topk(scores, cmax.reshape(B, NCH // 128, 128))
    return top[:, 0, :NUM_ITEMS]
